# baseline (speedup 1.0000x reference)
import functools

import jax
import jax.numpy as jnp
from jax import lax
from jax.experimental import pallas as pl
from jax.experimental.pallas import tpu as pltpu
from jax.experimental.pallas import tpu_sc as plsc

N_ROWS = 1000000
N_WORKERS = 32
CHUNK = 160
N_CHUNKS = N_ROWS // CHUNK
LANES = 16


def _sc_body(lt_hbm, x_hbm, out_hbm, ltv, xv, outv, gsem, ssem):
    wid = lax.axis_index("s") * 2 + lax.axis_index("c")
    nloc = (N_CHUNKS - 1 - wid) // N_WORKERS + 1

    def rows_of(k):
        ci = wid + k * N_WORKERS
        return pl.ds(ci * CHUNK, CHUNK)

    def start_gathers(k, b):
        rows = rows_of(k)
        pltpu.make_async_copy(lt_hbm.at[rows], ltv.at[b], gsem.at[0, b]).start()
        pltpu.make_async_copy(x_hbm.at[rows], xv.at[b], gsem.at[1, b]).start()

    def wait_gathers(b):
        pltpu.make_async_copy(lt_hbm.at[rows_of(0)], ltv.at[b], gsem.at[0, b]).wait()
        pltpu.make_async_copy(x_hbm.at[rows_of(0)], xv.at[b], gsem.at[1, b]).wait()

    def start_scatter(k, b):
        pltpu.make_async_copy(outv.at[b], out_hbm.at[rows_of(k)], ssem.at[b]).start()

    def wait_scatter(b):
        pltpu.make_async_copy(outv.at[b], out_hbm.at[rows_of(0)], ssem.at[b]).wait()

    start_gathers(0, 0)

    def step(k, carry):
        b = lax.rem(k, 2)

        @pl.when(k + 1 < nloc)
        def _():
            start_gathers(k + 1, 1 - b)

        wait_gathers(b)

        @pl.when(k >= 2)
        def _():
            wait_scatter(b)

        def row_interleave(r, c):
            for j in range(4):
                outv[b, r, pl.ds(16 * j, 16)] = ltv[b, r, pl.ds(16 * j, 16)]
                outv[b, r, pl.ds(64 + 16 * j, 16)] = xv[b, r, pl.ds(16 * j, 16)]
            return c

        lax.fori_loop(0, CHUNK, row_interleave, 0)
        start_scatter(k, b)
        return carry

    lax.fori_loop(0, nloc, step, 0)

    @pl.when(nloc >= 2)
    def _():
        wait_scatter(lax.rem(nloc, 2))

    wait_scatter(lax.rem(nloc + 1, 2))


def kernel(x, adj, lt):
    del adj
    n = lt.shape[0]
    mesh = plsc.VectorSubcoreMesh(core_axis_name="c", subcore_axis_name="s")
    sc_call = functools.partial(
        pl.kernel,
        mesh=mesh,
        out_type=jax.ShapeDtypeStruct((n, 128), jnp.float32),
        scratch_types=[
            pltpu.VMEM((2, CHUNK, 64), jnp.float32),
            pltpu.VMEM((2, CHUNK, 64), jnp.float32),
            pltpu.VMEM((2, CHUNK, 128), jnp.float32),
            pltpu.SemaphoreType.DMA((2, 2)),
            pltpu.SemaphoreType.DMA((2,)),
        ],
    )(_sc_body)
    return sc_call(lt, x)


# speedup vs baseline: 1.4871x; 1.4871x over previous
import jax
import jax.numpy as jnp
from jax.experimental import pallas as pl

N_ROWS = 1000000
BLOCK_ROWS = 20000


def _concat_body(lt_ref, x_ref, out_ref):
    out_ref[:, 0:64] = lt_ref[...]
    out_ref[:, 64:128] = x_ref[...]


def kernel(x, adj, lt):
    del adj
    n = lt.shape[0]
    grid = (n // BLOCK_ROWS,)
    return pl.pallas_call(
        _concat_body,
        grid=grid,
        in_specs=[
            pl.BlockSpec((BLOCK_ROWS, 64), lambda i: (i, 0)),
            pl.BlockSpec((BLOCK_ROWS, 64), lambda i: (i, 0)),
        ],
        out_specs=pl.BlockSpec((BLOCK_ROWS, 128), lambda i: (i, 0)),
        out_shape=jax.ShapeDtypeStruct((n, 128), jnp.float32),
    )(lt, x)
